# Initial kernel scaffold; baseline (speedup 1.0000x reference)
import jax
import jax.numpy as jnp
from jax.experimental import pallas as pl


def kernel(features, points_f, points_c, neighbors_f, pools, neighbors_c, upsamples, kp1, W1, kp2, W2, Wb, bb, Wg, bg, Wp, bp, Ws, bs, Wd, bd):
    raise NotImplementedError("write your pallas kernel here")


# speedup vs baseline: 3.9336x; 3.9336x over previous
import jax
import jax.numpy as jnp
from jax import lax
from jax.experimental import pallas as pl
from jax.experimental.pallas import tpu as pltpu
from jax.experimental.pallas import tpu_sc as plsc

N_F = 10000
N_C = 2500
N_CP = 2560
K_NB = 32
KP = 15
DGCNN_K = 10
D1 = 128
D2 = 256
DG = 256
DOUT = 32
SIGMA1 = 0.0625
SIGMA2 = 0.125

_NC_SC = 2
_NS_SC = 16
_NW = _NC_SC * _NS_SC


def _sc_gather_rows(table, idx, cs):
    B = idx.shape[0]
    D = table.shape[1]
    bpw = B // _NW
    nch = bpw // cs
    mesh = plsc.VectorSubcoreMesh(core_axis_name="c", subcore_axis_name="s")

    def body(table_hbm, idx_hbm, out_hbm, idx_v, rows_v, sem):
        wid = lax.axis_index("s") * _NC_SC + lax.axis_index("c")
        base = wid * bpw
        for j in range(nch):
            off = base + j * cs
            pltpu.sync_copy(idx_hbm.at[pl.ds(off, cs)], idx_v)
            pltpu.async_copy(table_hbm.at[idx_v], rows_v, sem).wait()
            pltpu.sync_copy(rows_v, out_hbm.at[pl.ds(off, cs)])

    f = pl.kernel(
        body,
        mesh=mesh,
        out_type=jax.ShapeDtypeStruct((B, D), jnp.float32),
        scratch_types=[
            pltpu.VMEM((cs,), jnp.int32),
            pltpu.VMEM((cs, D), jnp.float32),
            pltpu.SemaphoreType.DMA,
        ],
    )
    return f(table, idx)


def _sc_gather_planes(tab4, idx):
    V4 = tab4.shape[0]
    V = V4 // 4
    B = idx.shape[0]
    bpw = B // _NW
    nv = bpw // 16
    mesh = plsc.VectorSubcoreMesh(core_axis_name="c", subcore_axis_name="s")

    def body(tab_hbm, idx_hbm, o0, o1, o2, o3, tab_v, idx_v, v0, v1, v2, v3):
        wid = lax.axis_index("s") * _NC_SC + lax.axis_index("c")
        base = wid * bpw
        pltpu.sync_copy(tab_hbm, tab_v)
        pltpu.sync_copy(idx_hbm.at[pl.ds(base, bpw)], idx_v)
        outs_v = (v0, v1, v2, v3)

        def step(i, carry):
            o = pl.multiple_of(i * 16, 16)
            idx16 = idx_v[pl.ds(o, 16)]
            for p in range(4):
                outs_v[p][pl.ds(o, 16)] = plsc.load_gather(
                    tab_v, [idx16 + p * V])
            return carry

        lax.fori_loop(0, nv, step, 0)
        for p, o_hbm in enumerate((o0, o1, o2, o3)):
            pltpu.sync_copy(outs_v[p], o_hbm.at[pl.ds(base, bpw)])

    f = pl.kernel(
        body,
        mesh=mesh,
        out_type=tuple(jax.ShapeDtypeStruct((B,), jnp.float32)
                       for _ in range(4)),
        scratch_types=[
            pltpu.VMEM((V4,), jnp.float32),
            pltpu.VMEM((bpw,), jnp.int32),
            pltpu.VMEM((bpw,), jnp.float32),
            pltpu.VMEM((bpw,), jnp.float32),
            pltpu.VMEM((bpw,), jnp.float32),
            pltpu.VMEM((bpw,), jnp.float32),
        ],
        compiler_params=pltpu.CompilerParams(needs_layout_passes=False),
    )
    return f(tab4, idx)


def _influence(nx, ny, nz, q_ref, kp_ref, inv_sigma):
    kx = kp_ref[0:1, :][:, None, :]
    ky = kp_ref[1:2, :][:, None, :]
    kz = kp_ref[2:3, :][:, None, :]
    dx = (nx - q_ref[:, 0:1])[:, :, None] - kx
    dy = (ny - q_ref[:, 1:2])[:, :, None] - ky
    dz = (nz - q_ref[:, 2:3])[:, :, None] - kz
    d = jnp.sqrt(dx * dx + dy * dy + dz * dz)
    return jnp.maximum(0.0, 1.0 - d * inv_sigma)


def _tc_kpconv1(g1x, g1y, g1z, g1f, pf, kp16, w1e, bn):
    def body(x_ref, y_ref, z_ref, f_ref, q_ref, kp_ref, w_ref, o_ref):
        infl = _influence(x_ref[...], y_ref[...], z_ref[...], q_ref,
                          kp_ref, 1.0 / SIGMA1)
        s1 = jnp.sum(infl * f_ref[...][:, :, None], axis=1)
        f1 = jnp.dot(s1, w_ref[...], preferred_element_type=jnp.float32)
        o_ref[...] = jnp.maximum(f1, 0.0)

    nb = pl.BlockSpec((bn, K_NB), lambda i: (i, 0))
    return pl.pallas_call(
        body,
        grid=(N_F // bn,),
        in_specs=[
            nb, nb, nb, nb,
            pl.BlockSpec((bn, 3), lambda i: (i, 0)),
            pl.BlockSpec((8, 16), lambda i: (0, 0)),
            pl.BlockSpec((16, D1), lambda i: (0, 0)),
        ],
        out_specs=pl.BlockSpec((bn, D1), lambda i: (i, 0)),
        out_shape=jax.ShapeDtypeStruct((N_F, D1), jnp.float32),
    )(g1x, g1y, g1z, g1f, pf, kp16, w1e)


def _tc_kpconv2(g2f, g2x, g2y, g2z, pc, kp16, w2, wb, bb, wgb, bc):
    def body(gf_ref, x_ref, y_ref, z_ref, q_ref, kp_ref, w2_ref, wb_ref,
             bb_ref, wgb_ref, fb_ref, fbp_ref):
        infl = _influence(x_ref[...], y_ref[...], z_ref[...], q_ref,
                          kp_ref, 1.0 / SIGMA2)
        gf = gf_ref[...]
        acc = jnp.zeros((bc, D2), dtype=jnp.float32)
        for p in range(KP):
            wp = jnp.sum(gf * infl[:, :, p:p + 1], axis=1)
            acc = acc + jnp.dot(wp, w2_ref[p],
                                preferred_element_type=jnp.float32)
        f2 = jnp.maximum(acc, 0.0)
        fb = (jnp.dot(f2, wb_ref[...], preferred_element_type=jnp.float32)
              + bb_ref[0:1, :])
        fb_ref[...] = fb
        fbp_ref[...] = jnp.dot(fb, wgb_ref[...],
                               preferred_element_type=jnp.float32)

    nb = pl.BlockSpec((bc, K_NB), lambda i: (i, 0))
    return pl.pallas_call(
        body,
        grid=(N_CP // bc,),
        in_specs=[
            pl.BlockSpec((bc, K_NB, D1), lambda i: (i, 0, 0)),
            nb, nb, nb,
            pl.BlockSpec((bc, 3), lambda i: (i, 0)),
            pl.BlockSpec((8, 16), lambda i: (0, 0)),
            pl.BlockSpec((KP, D1, D2), lambda i: (0, 0, 0)),
            pl.BlockSpec((D2, DG), lambda i: (0, 0)),
            pl.BlockSpec((8, DG), lambda i: (0, 0)),
            pl.BlockSpec((DG, DG), lambda i: (0, 0)),
        ],
        out_specs=[
            pl.BlockSpec((bc, DG), lambda i: (i, 0)),
            pl.BlockSpec((bc, DG), lambda i: (i, 0)),
        ],
        out_shape=[
            jax.ShapeDtypeStruct((N_CP, DG), jnp.float32),
            jax.ShapeDtypeStruct((N_CP, DG), jnp.float32),
        ],
    )(g2f, g2x, g2y, g2z, pc, kp16, w2, wb, bb, wgb)


def _tc_edge_proj(fb, fbp, g3, wgt, bg, wp, bp, ws16, bs16, wd0p, wds, wd1,
                  bc):
    def body(fb_ref, fbp_ref, nb_ref, wgt_ref, bg_ref, wp_ref, bp_ref,
             ws_ref, bs_ref, wd0_ref, wds_ref, wd1_ref, o_ref):
        ctr = fb_ref[...]
        own = fbp_ref[...]
        hc = (jnp.dot(ctr, wgt_ref[...], preferred_element_type=jnp.float32)
              + bg_ref[0:1, :] - own)
        mx = jnp.zeros((bc, DG), dtype=jnp.float32)
        for k in range(DGCNN_K):
            hk = hc + nb_ref[:, k, :]
            mx = jnp.maximum(mx, jnp.maximum(hk, 0.0))
        fg = ctr + mx
        fp = (jnp.dot(fg, wp_ref[...], preferred_element_type=jnp.float32)
              + bp_ref[0:1, :])
        nrm = jnp.sqrt(jnp.sum(fp * fp, axis=1, keepdims=True)) + 1e-8
        fp = fp / nrm
        s = (jnp.dot(fg, ws_ref[...], preferred_element_type=jnp.float32)
             [:, 0:1] + bs_ref[0:1, 0:1])
        bad = jnp.logical_or(jnp.isnan(s), jnp.isinf(s))
        s = jnp.where(bad, 0.0, s)
        d4 = (jnp.dot(fp, wd0_ref[...], preferred_element_type=jnp.float32)
              + s * wds_ref[0:1, :] + wd1_ref[0:1, :])
        o_ref[...] = d4

    return pl.pallas_call(
        body,
        grid=(N_CP // bc,),
        in_specs=[
            pl.BlockSpec((bc, DG), lambda i: (i, 0)),
            pl.BlockSpec((bc, DG), lambda i: (i, 0)),
            pl.BlockSpec((bc, DGCNN_K, DG), lambda i: (i, 0, 0)),
            pl.BlockSpec((DG, DG), lambda i: (0, 0)),
            pl.BlockSpec((8, DG), lambda i: (0, 0)),
            pl.BlockSpec((DG, DG), lambda i: (0, 0)),
            pl.BlockSpec((8, DG), lambda i: (0, 0)),
            pl.BlockSpec((DG, 16), lambda i: (0, 0)),
            pl.BlockSpec((8, 16), lambda i: (0, 0)),
            pl.BlockSpec((DG, D1), lambda i: (0, 0)),
            pl.BlockSpec((8, D1), lambda i: (0, 0)),
            pl.BlockSpec((8, D1), lambda i: (0, 0)),
        ],
        out_specs=pl.BlockSpec((bc, D1), lambda i: (i, 0)),
        out_shape=jax.ShapeDtypeStruct((N_CP, D1), jnp.float32),
    )(fb, fbp, g3, wgt, bg, wp, bp, ws16, bs16, wd0p, wds, wd1)


def _tc_decoder(g4, f1, wdf, bd, bn):
    def body(g_ref, f1_ref, wdf_ref, bd_ref, o_ref):
        o = (g_ref[:, 0:DOUT]
             + jnp.dot(f1_ref[...], wdf_ref[...],
                       preferred_element_type=jnp.float32)
             + bd_ref[0:1, :])
        o_ref[...] = jnp.maximum(o, 0.0)

    return pl.pallas_call(
        body,
        grid=(N_F // bn,),
        in_specs=[
            pl.BlockSpec((bn, D1), lambda i: (i, 0)),
            pl.BlockSpec((bn, D1), lambda i: (i, 0)),
            pl.BlockSpec((D1, DOUT), lambda i: (0, 0)),
            pl.BlockSpec((8, DOUT), lambda i: (0, 0)),
        ],
        out_specs=pl.BlockSpec((bn, DOUT), lambda i: (i, 0)),
        out_shape=jax.ShapeDtypeStruct((N_F, DOUT), jnp.float32),
    )(g4, f1, wdf, bd)


def _row8(v, w):
    out = jnp.zeros((8, w), dtype=jnp.float32)
    return out.at[0, :v.shape[0]].set(v)


def _kp16(kp):
    out = jnp.zeros((8, 16), dtype=jnp.float32)
    return out.at[0:3, 0:KP].set(kp.T)


def kernel(features, points_f, points_c, neighbors_f, pools, neighbors_c,
           upsamples, kp1, W1, kp2, W2, Wb, bb, Wg, bg, Wp, bp, Ws, bs,
           Wd, bd):
    f32 = jnp.float32
    tab4 = jnp.concatenate(
        [points_f[:, 0], points_f[:, 1], points_f[:, 2], features[:, 0]])
    pc3 = jnp.pad(points_c, ((0, N_CP - N_C), (0, 0)))

    w1e = jnp.pad(W1[:, 0, :], ((0, 1), (0, 0)))
    wgt = Wg[:DG]
    wgb = Wg[DG:]
    ws16 = jnp.zeros((DG, 16), dtype=f32).at[:, 0:1].set(Ws)
    wd0p = jnp.zeros((DG, D1), dtype=f32).at[:, :DOUT].set(Wd[:DG])
    wds = _row8(Wd[DG], D1)
    wd1 = _row8(Wd[DG + 1], D1)
    wdf = Wd[DG + 2:]

    idx1 = neighbors_f.reshape(-1).astype(jnp.int32)
    g1x, g1y, g1z, g1f = _sc_gather_planes(tab4, idx1)
    shp1 = (N_F, K_NB)
    f1 = _tc_kpconv1(g1x.reshape(shp1), g1y.reshape(shp1),
                     g1z.reshape(shp1), g1f.reshape(shp1),
                     points_f, _kp16(kp1), w1e, bn=1000)

    idx2 = jnp.pad(pools.reshape(-1).astype(jnp.int32),
                   (0, (N_CP - N_C) * K_NB))
    g2f = _sc_gather_rows(f1, idx2, cs=640).reshape(N_CP, K_NB, D1)
    g2x, g2y, g2z, _ = _sc_gather_planes(tab4, idx2)
    shp2 = (N_CP, K_NB)
    fb, fbp = _tc_kpconv2(g2f, g2x.reshape(shp2), g2y.reshape(shp2),
                          g2z.reshape(shp2), pc3, _kp16(kp2), W2, Wb,
                          _row8(bb, DG), wgb, bc=320)

    idx3 = jnp.pad(neighbors_c.reshape(-1).astype(jnp.int32),
                   (0, (N_CP - N_C) * DGCNN_K))
    g3 = _sc_gather_rows(fbp, idx3, cs=400).reshape(N_CP, DGCNN_K, DG)
    d4 = _tc_edge_proj(fb, fbp, g3, wgt, _row8(bg, DG), Wp, _row8(bp, DG),
                       ws16, _row8(bs, 16), wd0p, wds, wd1,
                       bc=512)

    idx4 = jnp.pad(upsamples[:, 0].astype(jnp.int32), (0, 10240 - N_F))
    g4 = _sc_gather_rows(d4, idx4, cs=320)[:N_F]
    out = _tc_decoder(g4, f1, wdf, _row8(bd, DOUT), bn=1000)
    return out
